# Initial kernel scaffold; baseline (speedup 1.0000x reference)
import jax
import jax.numpy as jnp
from jax.experimental import pallas as pl


def kernel(seq, token_table, pos_table):
    raise NotImplementedError("write your pallas kernel here")


# speedup vs baseline: 1.2934x; 1.2934x over previous
import jax
import jax.numpy as jnp
from jax import lax
from jax.experimental import pallas as pl
from jax.experimental.pallas import tpu as pltpu
from jax.experimental.pallas import tpu_sc as plsc

BATCH = 4096
SEQ_LEN = 200
DEPTH = 32
NW = 32
ROWS_PER_W = BATCH * SEQ_LEN // NW
GROUP = 800
N_GROUPS = ROWS_PER_W // GROUP


def _sc_body(seq_hbm, pos4_hbm, table_hbm, out_hbm, idx_v, rows_v, pos_v, sem):
    wid = lax.axis_index("s") * 2 + lax.axis_index("c")
    base = wid * ROWS_PER_W
    pltpu.sync_copy(pos4_hbm, pos_v)

    def group_body(g, carry):
        start = base + g * GROUP
        pltpu.sync_copy(seq_hbm.at[pl.ds(start, GROUP)], idx_v)
        pltpu.async_copy(table_hbm.at[idx_v], rows_v, sem).wait()

        def add_body(i, c):
            rows_v[i, pl.ds(0, 16)] += pos_v[i, pl.ds(0, 16)]
            rows_v[i, pl.ds(16, 16)] += pos_v[i, pl.ds(16, 16)]
            return c

        lax.fori_loop(0, GROUP, add_body, 0)
        pltpu.sync_copy(rows_v, out_hbm.at[pl.ds(start, GROUP)])
        return carry

    lax.fori_loop(0, N_GROUPS, group_body, 0)


def kernel(seq, token_table, pos_table):
    seq_flat = seq.reshape(-1).astype(jnp.int32)
    pos4 = jnp.tile(pos_table, (GROUP // SEQ_LEN, 1))
    mesh = plsc.VectorSubcoreMesh(core_axis_name="c", subcore_axis_name="s")
    out = pl.kernel(
        _sc_body,
        out_type=jax.ShapeDtypeStruct((BATCH * SEQ_LEN, DEPTH), jnp.float32),
        mesh=mesh,
        compiler_params=pltpu.CompilerParams(use_tc_tiling_on_sc=False),
        scratch_types=[
            pltpu.VMEM((GROUP,), jnp.int32),
            pltpu.VMEM((GROUP, DEPTH), jnp.float32),
            pltpu.VMEM((GROUP, DEPTH), jnp.float32),
            pltpu.SemaphoreType.DMA,
        ],
    )(seq_flat, pos4, token_table)
    return out.reshape(BATCH, SEQ_LEN, DEPTH)


# speedup vs baseline: 1.4914x; 1.1531x over previous
import jax
import jax.numpy as jnp
from jax import lax
from jax.experimental import pallas as pl
from jax.experimental.pallas import tpu as pltpu
from jax.experimental.pallas import tpu_sc as plsc

BATCH = 4096
SEQ_LEN = 200
DEPTH = 32
NW = 32
ROWS_PER_W = BATCH * SEQ_LEN // NW
GROUP = 800
N_GROUPS = ROWS_PER_W // GROUP
SEQS_PER_GROUP = GROUP // SEQ_LEN


def _sc_body(seq_hbm, pos_hbm, table_hbm, out_hbm,
             idx_v, rows0, rows1, pos_v, gsem0, gsem1, osem0, osem1):
    wid = lax.axis_index("s") * 2 + lax.axis_index("c")
    base = wid * ROWS_PER_W
    pltpu.sync_copy(seq_hbm.at[wid], idx_v)
    pltpu.sync_copy(pos_hbm, pos_v)

    def gather(g, rows_ref, sem):
        pltpu.async_copy(table_hbm.at[idx_v.at[g]], rows_ref, sem)

    def gather_wait(rows_ref, sem):
        pltpu.make_async_copy(table_hbm.at[idx_v.at[0]], rows_ref, sem).wait()

    def store(g, rows_ref, sem):
        pltpu.async_copy(rows_ref, out_hbm.at[pl.ds(base + g * GROUP, GROUP)],
                         sem)

    def store_wait(rows_ref, sem):
        pltpu.make_async_copy(rows_ref, out_hbm.at[pl.ds(base, GROUP)],
                              sem).wait()

    def add_pos(rows_ref):
        @pl.loop(0, SEQ_LEN, unroll=2)
        def _(l):
            p0 = pos_v[l, pl.ds(0, 16)]
            p1 = pos_v[l, pl.ds(16, 16)]
            for r in range(SEQS_PER_GROUP):
                plsc.addupdate(rows_ref.at[l + r * SEQ_LEN, pl.ds(0, 16)], p0)
                plsc.addupdate(rows_ref.at[l + r * SEQ_LEN, pl.ds(16, 16)], p1)

    gather(0, rows0, gsem0)

    def pipe_body(i, carry):
        g0 = 2 * i

        @pl.when(g0 > 0)
        def _():
            store_wait(rows1, osem1)
        gather(g0 + 1, rows1, gsem1)

        gather_wait(rows0, gsem0)
        add_pos(rows0)
        store(g0, rows0, osem0)

        @pl.when(g0 + 2 < N_GROUPS)
        def _():
            store_wait(rows0, osem0)
            gather(g0 + 2, rows0, gsem0)

        gather_wait(rows1, gsem1)
        add_pos(rows1)
        store(g0 + 1, rows1, osem1)
        return carry

    lax.fori_loop(0, N_GROUPS // 2, pipe_body, 0)
    store_wait(rows0, osem0)
    store_wait(rows1, osem1)


def kernel(seq, token_table, pos_table):
    seq_w = seq.reshape(NW, N_GROUPS, GROUP).astype(jnp.int32)
    mesh = plsc.VectorSubcoreMesh(core_axis_name="c", subcore_axis_name="s")
    out = pl.kernel(
        _sc_body,
        out_type=jax.ShapeDtypeStruct((BATCH * SEQ_LEN, DEPTH), jnp.float32),
        mesh=mesh,
        compiler_params=pltpu.CompilerParams(use_tc_tiling_on_sc=False),
        scratch_types=[
            pltpu.VMEM((ROWS_PER_W // GROUP, GROUP), jnp.int32),
            pltpu.VMEM((GROUP, DEPTH), jnp.float32),
            pltpu.VMEM((GROUP, DEPTH), jnp.float32),
            pltpu.VMEM((SEQ_LEN, DEPTH), jnp.float32),
            pltpu.SemaphoreType.DMA,
            pltpu.SemaphoreType.DMA,
            pltpu.SemaphoreType.DMA,
            pltpu.SemaphoreType.DMA,
        ],
    )(seq_w, pos_table, token_table)
    return out.reshape(BATCH, SEQ_LEN, DEPTH)
